# Initial kernel scaffold; baseline (speedup 1.0000x reference)
import jax
import jax.numpy as jnp
import numpy as np
from jax import lax
from jax.experimental import pallas as pl
from jax.experimental.pallas import tpu as pltpu
from jax.experimental.pallas import tpu_sc as plsc

_NUM_NODES = 100000
_NUM_EDGES = 3200000
_NUM_POSSIBLE = _NUM_NODES**2 - _NUM_NODES
_NUM_NONEDGES = _NUM_POSSIBLE - _NUM_EDGES
_EPS = float(-np.log(1.0 - _NUM_EDGES / _NUM_POSSIBLE))
_NEG_SCALE = float(_NUM_NONEDGES) / float(_NUM_EDGES)

_NC, _NS, _L = 2, 16, 16
_NW = _NC * _NS
_B = 262144
_TOT = 2 * _B
_PER_W = _TOT // _NW
_CHUNK = 128
_NCHUNK = _PER_W // _CHUNK
_D = 128
_KD = _D // _L


def _sc_dots_body(emb_hbm, li_hbm, ri_hbm, dots_hbm,
                  idx_l, idx_r, rows_l, rows_r, part, dots_v, sem):
    wid = lax.axis_index("s") * _NC + lax.axis_index("c")
    base = wid * _PER_W

    @pl.loop(0, _NCHUNK)
    def _chunk(c):
        off = base + c * _CHUNK
        pltpu.sync_copy(li_hbm.at[pl.ds(off, _CHUNK)], idx_l)
        pltpu.sync_copy(ri_hbm.at[pl.ds(off, _CHUNK)], idx_r)
        cl = pltpu.async_copy(emb_hbm.at[idx_l], rows_l, sem)
        cr = pltpu.async_copy(emb_hbm.at[idx_r], rows_r, sem)
        cl.wait()
        cr.wait()

        @pl.loop(0, _CHUNK)
        def _pair(p):
            acc0 = rows_l[p, pl.ds(0, _L)] * rows_r[p, pl.ds(0, _L)]
            acc1 = rows_l[p, pl.ds(_L, _L)] * rows_r[p, pl.ds(_L, _L)]
            for k in range(2, _KD):
                s = pl.ds(k * _L, _L)
                if k % 2 == 0:
                    acc0 = acc0 + rows_l[p, s] * rows_r[p, s]
                else:
                    acc1 = acc1 + rows_l[p, s] * rows_r[p, s]
            part[p, :] = acc0 + acc1

        @pl.loop(0, _CHUNK // _L)
        def _grp(g):
            row = g * _L + lax.iota(jnp.int32, _L)
            s = plsc.load_gather(part, [row, jnp.zeros((_L,), jnp.int32)])
            for j in range(1, _L):
                s = s + plsc.load_gather(part, [row, jnp.full((_L,), j, jnp.int32)])
            dots_v[pl.ds(g * _L, _L)] = s

        pltpu.sync_copy(dots_v, dots_hbm.at[pl.ds(off, _CHUNK)])


def _sc_dots(emb, li, ri):
    mesh = plsc.VectorSubcoreMesh(core_axis_name="c", subcore_axis_name="s",
                                  num_cores=_NC, num_subcores=_NS)
    return pl.kernel(
        _sc_dots_body,
        out_type=jax.ShapeDtypeStruct((_TOT,), jnp.float32),
        mesh=mesh,
        scratch_types=[
            pltpu.VMEM((_CHUNK,), jnp.int32),
            pltpu.VMEM((_CHUNK,), jnp.int32),
            pltpu.VMEM((_CHUNK, _D), jnp.float32),
            pltpu.VMEM((_CHUNK, _D), jnp.float32),
            pltpu.VMEM((_CHUNK, _L), jnp.float32),
            pltpu.VMEM((_CHUNK,), jnp.float32),
            pltpu.SemaphoreType.DMA,
        ],
    )(emb, li, ri)


def _loss_tc_body(e_ref, z_ref, o_ref):
    e = e_ref[...]
    z = z_ref[...]
    loss_edges = -jnp.mean(jnp.log(-jnp.expm1(-_EPS - e)))
    loss_non = jnp.mean(z)
    o_ref[0, 0] = (loss_edges + _NEG_SCALE * loss_non) / (1.0 + _NEG_SCALE)


def _loss_tc(edge_dots, non_dots):
    return pl.pallas_call(
        _loss_tc_body,
        out_shape=jax.ShapeDtypeStruct((1, 1), jnp.float32),
        out_specs=pl.BlockSpec(memory_space=pltpu.SMEM),
    )(edge_dots, non_dots)


@jax.jit
def kernel(emb, ones_idx, zeros_idx):
    li = jnp.concatenate([ones_idx[:, 0], zeros_idx[:, 0]]).astype(jnp.int32)
    ri = jnp.concatenate([ones_idx[:, 1], zeros_idx[:, 1]]).astype(jnp.int32)
    dots = _sc_dots(emb, li, ri)
    edge_dots = dots[:_B].reshape(_B // _D, _D)
    non_dots = dots[_B:].reshape(_B // _D, _D)
    return _loss_tc(edge_dots, non_dots)[0, 0]


# speedup vs baseline: 5.3615x; 5.3615x over previous
import jax
import jax.numpy as jnp
import numpy as np
from jax import lax
from jax.experimental import pallas as pl
from jax.experimental.pallas import tpu as pltpu
from jax.experimental.pallas import tpu_sc as plsc

_NUM_NODES = 100000
_NUM_EDGES = 3200000
_NUM_POSSIBLE = _NUM_NODES**2 - _NUM_NODES
_NUM_NONEDGES = _NUM_POSSIBLE - _NUM_EDGES
_EPS = float(-np.log(1.0 - _NUM_EDGES / _NUM_POSSIBLE))
_NEG_SCALE = float(_NUM_NONEDGES) / float(_NUM_EDGES)

_NC, _NS, _L = 2, 16, 16
_NW = _NC * _NS
_B = 262144
_TOT = 2 * _B
_PER_W = _TOT // _NW
_CHUNK = 128
_NCHUNK = _PER_W // _CHUNK
_D = 128
_KD = _D // _L


def _sc_dots_body(emb_hbm, li_hbm, ri_hbm, dots_hbm,
                  idx_l, idx_r, rows_l, rows_r, part, dots_v, sem):
    wid = lax.axis_index("s") * _NC + lax.axis_index("c")
    base = wid * _PER_W

    @pl.loop(0, _NCHUNK)
    def _chunk(c):
        off = base + c * _CHUNK
        pltpu.sync_copy(li_hbm.at[pl.ds(off, _CHUNK)], idx_l)
        pltpu.sync_copy(ri_hbm.at[pl.ds(off, _CHUNK)], idx_r)
        cl = pltpu.async_copy(emb_hbm.at[idx_l], rows_l, sem)
        cr = pltpu.async_copy(emb_hbm.at[idx_r], rows_r, sem)
        cl.wait()
        cr.wait()

        @pl.loop(0, _CHUNK)
        def _pair(p):
            acc0 = rows_l[p, pl.ds(0, _L)] * rows_r[p, pl.ds(0, _L)]
            acc1 = rows_l[p, pl.ds(_L, _L)] * rows_r[p, pl.ds(_L, _L)]
            for k in range(2, _KD):
                s = pl.ds(k * _L, _L)
                if k % 2 == 0:
                    acc0 = acc0 + rows_l[p, s] * rows_r[p, s]
                else:
                    acc1 = acc1 + rows_l[p, s] * rows_r[p, s]
            part[pl.ds(p * _L, _L)] = acc0 + acc1

        @pl.loop(0, _CHUNK // _L)
        def _grp(g):
            flat = (g * _L + lax.iota(jnp.int32, _L)) * _L
            s = plsc.load_gather(part, [flat])
            for j in range(1, _L):
                s = s + plsc.load_gather(part, [flat + j])
            dots_v[pl.ds(g * _L, _L)] = s

        pltpu.sync_copy(dots_v, dots_hbm.at[pl.ds(off, _CHUNK)])


def _sc_dots(emb, li, ri):
    mesh = plsc.VectorSubcoreMesh(core_axis_name="c", subcore_axis_name="s",
                                  num_cores=_NC, num_subcores=_NS)
    return pl.kernel(
        _sc_dots_body,
        out_type=jax.ShapeDtypeStruct((_TOT,), jnp.float32),
        mesh=mesh,
        scratch_types=[
            pltpu.VMEM((_CHUNK,), jnp.int32),
            pltpu.VMEM((_CHUNK,), jnp.int32),
            pltpu.VMEM((_CHUNK, _D), jnp.float32),
            pltpu.VMEM((_CHUNK, _D), jnp.float32),
            pltpu.VMEM((_CHUNK * _L,), jnp.float32),
            pltpu.VMEM((_CHUNK,), jnp.float32),
            pltpu.SemaphoreType.DMA,
        ],
        compiler_params=pltpu.CompilerParams(needs_layout_passes=False),
    )(emb, li, ri)


def _loss_tc_body(e_ref, z_ref, o_ref):
    e = e_ref[...]
    z = z_ref[...]
    loss_edges = -jnp.mean(jnp.log1p(-jnp.exp(-_EPS - e)))
    loss_non = jnp.mean(z)
    o_ref[0, 0] = (loss_edges + _NEG_SCALE * loss_non) / (1.0 + _NEG_SCALE)


def _loss_tc(edge_dots, non_dots):
    return pl.pallas_call(
        _loss_tc_body,
        out_shape=jax.ShapeDtypeStruct((1, 1), jnp.float32),
        out_specs=pl.BlockSpec(memory_space=pltpu.SMEM),
    )(edge_dots, non_dots)


@jax.jit
def kernel(emb, ones_idx, zeros_idx):
    li = jnp.concatenate([ones_idx[:, 0], zeros_idx[:, 0]]).astype(jnp.int32)
    ri = jnp.concatenate([ones_idx[:, 1], zeros_idx[:, 1]]).astype(jnp.int32)
    dots = _sc_dots(emb, li, ri)
    edge_dots = dots[:_B].reshape(_B // _D, _D)
    non_dots = dots[_B:].reshape(_B // _D, _D)
    return _loss_tc(edge_dots, non_dots)[0, 0]


# speedup vs baseline: 11.0873x; 2.0679x over previous
import jax
import jax.numpy as jnp
import numpy as np
from jax import lax
from jax.experimental import pallas as pl
from jax.experimental.pallas import tpu as pltpu
from jax.experimental.pallas import tpu_sc as plsc

_NUM_NODES = 100000
_NUM_EDGES = 3200000
_NUM_POSSIBLE = _NUM_NODES**2 - _NUM_NODES
_NUM_NONEDGES = _NUM_POSSIBLE - _NUM_EDGES
_EPS = float(-np.log(1.0 - _NUM_EDGES / _NUM_POSSIBLE))
_NEG_SCALE = float(_NUM_NONEDGES) / float(_NUM_EDGES)

_NC, _NS, _L = 2, 16, 16
_NW = _NC * _NS
_B = 262144
_TOT = 2 * _B
_PER_W = _TOT // _NW
_CHUNK = 128
_NCHUNK = _PER_W // _CHUNK
_D = 128
_KD = _D // _L


def _sc_dots_body(emb_hbm, li_hbm, ri_hbm, dots_hbm,
                  idx_l, idx_r, rows_l0, rows_r0, rows_l1, rows_r1,
                  part, dots_v, sem0, sem1):
    wid = lax.axis_index("s") * _NC + lax.axis_index("c")
    base = wid * _PER_W
    pltpu.sync_copy(li_hbm.at[pl.ds(base, _PER_W)], idx_l)
    pltpu.sync_copy(ri_hbm.at[pl.ds(base, _PER_W)], idx_r)

    def _issue(c, rl, rr, sem):
        pltpu.async_copy(emb_hbm.at[idx_l.at[pl.ds(c * _CHUNK, _CHUNK)]], rl, sem)
        pltpu.async_copy(emb_hbm.at[idx_r.at[pl.ds(c * _CHUNK, _CHUNK)]], rr, sem)

    def _wait(c, rl, rr, sem):
        pltpu.make_async_copy(
            emb_hbm.at[idx_l.at[pl.ds(c * _CHUNK, _CHUNK)]], rl, sem).wait()
        pltpu.make_async_copy(
            emb_hbm.at[idx_r.at[pl.ds(c * _CHUNK, _CHUNK)]], rr, sem).wait()

    def _compute(c, rl, rr):
        @pl.loop(0, _CHUNK)
        def _pair(p):
            acc0 = rl[p, pl.ds(0, _L)] * rr[p, pl.ds(0, _L)]
            acc1 = rl[p, pl.ds(_L, _L)] * rr[p, pl.ds(_L, _L)]
            for k in range(2, _KD):
                s = pl.ds(k * _L, _L)
                if k % 2 == 0:
                    acc0 = acc0 + rl[p, s] * rr[p, s]
                else:
                    acc1 = acc1 + rl[p, s] * rr[p, s]
            part[pl.ds(p * _L, _L)] = acc0 + acc1

        @pl.loop(0, _CHUNK // _L)
        def _grp(g):
            flat = (g * _L + lax.iota(jnp.int32, _L)) * _L
            s = plsc.load_gather(part, [flat])
            for j in range(1, _L):
                s = s + plsc.load_gather(part, [flat + j])
            dots_v[pl.ds(g * _L, _L)] = s

        pltpu.sync_copy(dots_v, dots_hbm.at[pl.ds(base + c * _CHUNK, _CHUNK)])

    _issue(0, rows_l0, rows_r0, sem0)

    @pl.loop(0, _NCHUNK, step=2)
    def _c(c):
        _issue(c + 1, rows_l1, rows_r1, sem1)
        _wait(c, rows_l0, rows_r0, sem0)
        _compute(c, rows_l0, rows_r0)

        @pl.when(c + 2 < _NCHUNK)
        def _refill():
            _issue(c + 2, rows_l0, rows_r0, sem0)

        _wait(c + 1, rows_l1, rows_r1, sem1)
        _compute(c + 1, rows_l1, rows_r1)


def _sc_dots(emb, li, ri):
    mesh = plsc.VectorSubcoreMesh(core_axis_name="c", subcore_axis_name="s",
                                  num_cores=_NC, num_subcores=_NS)
    return pl.kernel(
        _sc_dots_body,
        out_type=jax.ShapeDtypeStruct((_TOT,), jnp.float32),
        mesh=mesh,
        scratch_types=[
            pltpu.VMEM((_PER_W,), jnp.int32),
            pltpu.VMEM((_PER_W,), jnp.int32),
            pltpu.VMEM((_CHUNK, _D), jnp.float32),
            pltpu.VMEM((_CHUNK, _D), jnp.float32),
            pltpu.VMEM((_CHUNK, _D), jnp.float32),
            pltpu.VMEM((_CHUNK, _D), jnp.float32),
            pltpu.VMEM((_CHUNK * _L,), jnp.float32),
            pltpu.VMEM((_CHUNK,), jnp.float32),
            pltpu.SemaphoreType.DMA,
            pltpu.SemaphoreType.DMA,
        ],
        compiler_params=pltpu.CompilerParams(needs_layout_passes=False),
    )(emb, li, ri)


def _loss_tc_body(e_ref, z_ref, o_ref):
    e = e_ref[...]
    z = z_ref[...]
    loss_edges = -jnp.mean(jnp.log1p(-jnp.exp(-_EPS - e)))
    loss_non = jnp.mean(z)
    o_ref[0, 0] = (loss_edges + _NEG_SCALE * loss_non) / (1.0 + _NEG_SCALE)


def _loss_tc(edge_dots, non_dots):
    return pl.pallas_call(
        _loss_tc_body,
        out_shape=jax.ShapeDtypeStruct((1, 1), jnp.float32),
        out_specs=pl.BlockSpec(memory_space=pltpu.SMEM),
    )(edge_dots, non_dots)


@jax.jit
def kernel(emb, ones_idx, zeros_idx):
    li = jnp.concatenate([ones_idx[:, 0], zeros_idx[:, 0]]).astype(jnp.int32)
    ri = jnp.concatenate([ones_idx[:, 1], zeros_idx[:, 1]]).astype(jnp.int32)
    dots = _sc_dots(emb, li, ri)
    edge_dots = dots[:_B].reshape(_B // _D, _D)
    non_dots = dots[_B:].reshape(_B // _D, _D)
    return _loss_tc(edge_dots, non_dots)[0, 0]
